# Initial kernel scaffold; baseline (speedup 1.0000x reference)
import jax
import jax.numpy as jnp
from jax.experimental import pallas as pl


def kernel(x, edge_index, batch, params):
    raise NotImplementedError("write your pallas kernel here")


# speedup vs baseline: 6.6923x; 6.6923x over previous
import functools

import jax
import jax.numpy as jnp
from jax import lax
from jax.experimental import pallas as pl
from jax.experimental.pallas import tpu as pltpu
from jax.experimental.pallas import tpu_sc as plsc

_N = 50000
_NP = 50176
_E = 800000
_EP = 819200
_NG = 128
_RT = 256
_NT = _NP // _RT
_C = 32
_NW = 32
_EPW = _EP // _NW
_ECH = 512
_NST = _ECH // 128
_NCH = _EPW // _ECH
_RPS = _NP // 16
_ZR = 98
_EPS = 1e-5
_PREC = lax.Precision.HIGHEST


def _sc_aggregate(feat, src_r, dst_r):
    mesh = plsc.VectorSubcoreMesh(core_axis_name="c", subcore_axis_name="s")

    def body(feat_h, src_h, dst_h, out_h, srcv, dstv, rows, zbuf, acc, gsem):
        c = lax.axis_index("c")
        s = lax.axis_index("s")

        def zrow(i, _):
            zbuf[i, pl.ds(0, 16)] = jnp.zeros((16,), jnp.float32)
            zbuf[i, pl.ds(16, 16)] = jnp.zeros((16,), jnp.float32)
            return 0

        lax.fori_loop(0, _ZR, zrow, 0)

        base = s * _RPS

        def zchunk(i, _):
            pltpu.sync_copy(zbuf, acc.at[pl.ds(base + i * _ZR, _ZR)])
            return 0

        lax.fori_loop(0, _RPS // _ZR, zchunk, 0)
        plsc.subcore_barrier()

        wid = s * 2 + c
        row0 = wid * (_EPW // 128)

        def echunk(j, _):
            roff = row0 + j * _NST
            pltpu.sync_copy(src_h.at[pl.ds(roff, _NST)], srcv)
            pltpu.sync_copy(dst_h.at[pl.ds(roff, _NST)], dstv)
            descs = [
                pltpu.async_copy(
                    feat_h.at[srcv.at[k]], rows.at[pl.ds(k * 128, 128)], gsem
                )
                for k in range(_NST)
            ]
            for d in descs:
                d.wait()
            for k in range(_NST):
                pltpu.sync_copy(
                    rows.at[pl.ds(k * 128, 128)], acc.at[dstv.at[k]], add=True
                )
            return 0

        lax.fori_loop(0, _NCH, echunk, 0)
        plsc.subcore_barrier()

        pltpu.sync_copy(
            acc.at[pl.ds(base, _RPS)], out_h.at[pl.ds(c * _NP + base, _RPS)]
        )

    out = pl.kernel(
        body,
        out_type=jax.ShapeDtypeStruct((2 * _NP, _C), jnp.float32),
        mesh=mesh,
        compiler_params=pltpu.CompilerParams(use_tc_tiling_on_sc=False),
        scratch_types=[
            pltpu.VMEM((_NST, 128), jnp.int32),
            pltpu.VMEM((_NST, 128), jnp.int32),
            pltpu.VMEM((_ECH, _C), jnp.float32),
            pltpu.VMEM((_ZR, _C), jnp.float32),
            pltpu.VMEM_SHARED((_NP, _C), jnp.float32),
            pltpu.SemaphoreType.DMA,
        ],
    )(feat, src_r, dst_r)
    return out.reshape(2, _NP, _C)


def _conv_mm(pairs, feats, directs, bias, cout):
    n_p, n_f, n_d = len(pairs), len(feats), len(directs)

    def body(*refs):
        i = pl.program_id(0)
        it = iter(refs)
        p_refs = [(next(it), next(it)) for _ in range(n_p)]
        f_refs = [(next(it), next(it)) for _ in range(n_f)]
        d_refs = [next(it) for _ in range(n_d)]
        b_ref = next(it)
        y_ref, s_ref, ss_ref = next(it), next(it), next(it)

        y = jnp.broadcast_to(b_ref[...], (_RT, cout))
        for pr, wr in p_refs:
            y = y + jnp.dot(pr[0] + pr[1], wr[...],
                            preferred_element_type=jnp.float32,
                            precision=_PREC)
        for fr, wr in f_refs:
            y = y + jnp.dot(fr[...], wr[...],
                            preferred_element_type=jnp.float32,
                            precision=_PREC)
        for dr in d_refs:
            y = y + dr[0] + dr[1]

        rows = i * _RT + lax.broadcasted_iota(jnp.int32, (_RT, 1), 0)
        y = jnp.where(rows < _N, y, 0.0)
        y_ref[...] = y

        @pl.when(i == 0)
        def _():
            s_ref[...] = jnp.zeros_like(s_ref)
            ss_ref[...] = jnp.zeros_like(ss_ref)

        s_ref[...] += jnp.sum(y, axis=0, keepdims=True)
        ss_ref[...] += jnp.sum(y * y, axis=0, keepdims=True)

    in_specs = []
    args = []
    for p, w in pairs:
        in_specs.append(pl.BlockSpec((2, _RT, _C), lambda i: (0, i, 0)))
        args.append(p)
        in_specs.append(pl.BlockSpec(w.shape, lambda i: (0, 0)))
        args.append(w)
    for f, w in feats:
        ck = f.shape[1]
        in_specs.append(pl.BlockSpec((_RT, ck), lambda i: (i, 0)))
        args.append(f)
        in_specs.append(pl.BlockSpec(w.shape, lambda i: (0, 0)))
        args.append(w)
    for d in directs:
        in_specs.append(pl.BlockSpec((2, _RT, _C), lambda i: (0, i, 0)))
        args.append(d)
    in_specs.append(pl.BlockSpec((1, cout), lambda i: (0, 0)))
    args.append(bias.reshape(1, cout))

    return pl.pallas_call(
        body,
        grid=(_NT,),
        in_specs=in_specs,
        out_specs=[
            pl.BlockSpec((_RT, cout), lambda i: (i, 0)),
            pl.BlockSpec((1, cout), lambda i: (0, 0)),
            pl.BlockSpec((1, cout), lambda i: (0, 0)),
        ],
        out_shape=[
            jax.ShapeDtypeStruct((_NP, cout), jnp.float32),
            jax.ShapeDtypeStruct((1, cout), jnp.float32),
            jax.ShapeDtypeStruct((1, cout), jnp.float32),
        ],
    )(*args)


def _bn_coeffs(s_ref, ss_ref, g_ref, b_ref):
    mean = s_ref[...] * (1.0 / _N)
    var = jnp.maximum(ss_ref[...] * (1.0 / _N) - mean * mean, 0.0)
    scale = g_ref[...] * lax.rsqrt(var + _EPS)
    shift = b_ref[...] - mean * scale
    return scale, shift


def _bn_relu(y, s, ss, gamma, beta, nchunks=1, mask_col=None):
    c = y.shape[1]

    def body(y_ref, s_ref, ss_ref, g_ref, b_ref, *out_refs):
        i = pl.program_id(0)
        scale, shift = _bn_coeffs(s_ref, ss_ref, g_ref, b_ref)
        h = jnp.maximum(y_ref[...] * scale + shift, 0.0)
        rows = i * _RT + lax.broadcasted_iota(jnp.int32, (_RT, 1), 0)
        rmask = rows < _N
        h = jnp.where(rmask, h, 0.0)
        if mask_col is not None:
            cols = lax.broadcasted_iota(jnp.int32, (_RT, c), 1)
            h = h + jnp.where((cols == mask_col) & rmask, 1.0, 0.0)
        if nchunks == 1:
            out_refs[0][...] = h
        else:
            for k in range(nchunks):
                out_refs[k][...] = h[:, k * _C:(k + 1) * _C]

    if nchunks == 1:
        out_specs = [pl.BlockSpec((_RT, c), lambda i: (i, 0))]
        out_shape = [jax.ShapeDtypeStruct((_NP, c), jnp.float32)]
    else:
        out_specs = [pl.BlockSpec((_RT, _C), lambda i: (i, 0))
                     for _ in range(nchunks)]
        out_shape = [jax.ShapeDtypeStruct((_NP, _C), jnp.float32)
                     for _ in range(nchunks)]

    outs = pl.pallas_call(
        body,
        grid=(_NT,),
        in_specs=[
            pl.BlockSpec((_RT, c), lambda i: (i, 0)),
            pl.BlockSpec((1, c), lambda i: (0, 0)),
            pl.BlockSpec((1, c), lambda i: (0, 0)),
            pl.BlockSpec((1, c), lambda i: (0, 0)),
            pl.BlockSpec((1, c), lambda i: (0, 0)),
        ],
        out_specs=out_specs,
        out_shape=out_shape,
    )(y, s, ss, gamma.reshape(1, c), beta.reshape(1, c))
    return outs


def _bn_relu_mm(y, s, ss, gamma, beta, w):
    c = y.shape[1]
    cu = w.shape[1]

    def body(y_ref, s_ref, ss_ref, g_ref, b_ref, w_ref, t_ref, u_ref):
        i = pl.program_id(0)
        scale, shift = _bn_coeffs(s_ref, ss_ref, g_ref, b_ref)
        t = jnp.maximum(y_ref[...] * scale + shift, 0.0)
        rows = i * _RT + lax.broadcasted_iota(jnp.int32, (_RT, 1), 0)
        t = jnp.where(rows < _N, t, 0.0)
        t_ref[...] = t
        u_ref[...] = jnp.dot(t, w_ref[...],
                             preferred_element_type=jnp.float32,
                             precision=_PREC)

    return pl.pallas_call(
        body,
        grid=(_NT,),
        in_specs=[
            pl.BlockSpec((_RT, c), lambda i: (i, 0)),
            pl.BlockSpec((1, c), lambda i: (0, 0)),
            pl.BlockSpec((1, c), lambda i: (0, 0)),
            pl.BlockSpec((1, c), lambda i: (0, 0)),
            pl.BlockSpec((1, c), lambda i: (0, 0)),
            pl.BlockSpec((c, cu), lambda i: (0, 0)),
        ],
        out_specs=[
            pl.BlockSpec((_RT, c), lambda i: (i, 0)),
            pl.BlockSpec((_RT, cu), lambda i: (i, 0)),
        ],
        out_shape=[
            jax.ShapeDtypeStruct((_NP, c), jnp.float32),
            jax.ShapeDtypeStruct((_NP, cu), jnp.float32),
        ],
    )(y, s, ss, gamma.reshape(1, c), beta.reshape(1, c), w)


def _pool_cls(h, batch_r, cls_w, cls_b):
    cheq = h.shape[1]
    cw = cls_w.shape[1]

    def body(h_ref, b_ref, w_ref, bias_ref, o_ref, acc):
        i = pl.program_id(0)

        @pl.when(i == 0)
        def _():
            acc[...] = jnp.zeros_like(acc)

        bvec = b_ref[0, 0, :].reshape(_RT, 1)
        onehot = (bvec == lax.broadcasted_iota(jnp.int32, (_RT, _NG), 1))
        onehot = onehot.astype(jnp.float32)
        acc[...] += jnp.dot(onehot.T, h_ref[...],
                            preferred_element_type=jnp.float32,
                            precision=_PREC)

        @pl.when(i == _NT - 1)
        def _():
            a = acc[...]
            cnt = jnp.maximum(a[:, 200:201], 1.0)
            pooled = a[:, :200] / cnt
            o_ref[...] = jnp.dot(pooled, w_ref[...],
                                 preferred_element_type=jnp.float32,
                                 precision=_PREC) + bias_ref[...]

    return pl.pallas_call(
        body,
        grid=(_NT,),
        in_specs=[
            pl.BlockSpec((_RT, cheq), lambda i: (i, 0)),
            pl.BlockSpec((1, 1, _RT), lambda i: (i, 0, 0)),
            pl.BlockSpec(cls_w.shape, lambda i: (0, 0)),
            pl.BlockSpec((1, cw), lambda i: (0, 0)),
        ],
        out_specs=pl.BlockSpec((_NG, cw), lambda i: (0, 0)),
        out_shape=jax.ShapeDtypeStruct((_NG, cw), jnp.float32),
        scratch_shapes=[pltpu.VMEM((_NG, cheq), jnp.float32)],
    )(h, batch_r, cls_w, cls_b.reshape(1, cw))


def _pad_rows(wt, rows_to):
    out = jnp.zeros((rows_to, wt.shape[1]), jnp.float32)
    return out.at[: wt.shape[0]].set(wt)


def _split_weight(wt, widths):
    blocks = []
    off = 0
    for w in widths:
        blocks.append(_pad_rows(wt[off:off + w], _C))
        off += w
    return blocks


def _pad_cols(v, cols_to):
    out = jnp.zeros((cols_to,), jnp.float32)
    return out.at[: v.shape[0]].set(v)


def _pad_mat(wt, rows_to, cols_to):
    out = jnp.zeros((rows_to, cols_to), jnp.float32)
    return out.at[: wt.shape[0], : wt.shape[1]].set(wt)


def kernel(x, edge_index, batch, params):
    src = edge_index[0]
    dst = edge_index[1]
    npad = _EP - _E
    pad_iota = lax.iota(jnp.int32, npad)
    src_p = jnp.concatenate([src, pad_iota % _N])
    dst_p = jnp.concatenate([dst, _N + pad_iota % (_NP - _N)])
    src_r = src_p.reshape(_EP // 128, 128)
    dst_r = dst_p.reshape(_EP // 128, 128)

    x_pad = jnp.zeros((_NP, _C), jnp.float32).at[:_N, :22].set(x)
    batch_p = jnp.full((_NP,), _NG, jnp.int32).at[:_N].set(batch)
    batch_r = batch_p.reshape(_NT, 1, _RT)

    p0 = params["conv0"]
    px = _sc_aggregate(x_pad, src_r, dst_r)
    y, s, ss = _conv_mm(
        pairs=[(px, _pad_rows(p0["Wrel"].T, _C))],
        feats=[(x_pad, _pad_rows(p0["Wroot"].T, _C))],
        directs=[],
        bias=p0["brel"],
        cout=_C,
    )
    (h0,) = _bn_relu(y, s, ss, p0["gamma"], p0["beta"])
    feats = [h0]
    widths = [32]
    aggs = [_sc_aggregate(h0, src_r, dst_r)]

    for bi in range(2):
        for layer in params[f"block{bi + 1}"]:
            c1 = layer["conv1"]
            c2 = layer["conv2"]
            wa = _split_weight(c1["Wrel"].T, widths)
            wb = _split_weight(c1["Wroot"].T, widths)
            y1, s1, ss1 = _conv_mm(
                pairs=list(zip(aggs, wa)),
                feats=list(zip(feats, wb)),
                directs=[],
                bias=c1["brel"],
                cout=64,
            )
            t1, u2 = _bn_relu_mm(y1, s1, ss1, c1["gamma"], c1["beta"],
                                 c2["Wrel"].T)
            pu = _sc_aggregate(u2, src_r, dst_r)
            y2, s2, ss2 = _conv_mm(
                pairs=[],
                feats=[(t1, c2["Wroot"].T)],
                directs=[pu],
                bias=c2["brel"],
                cout=_C,
            )
            (t2,) = _bn_relu(y2, s2, ss2, c2["gamma"], c2["beta"])
            feats.append(t2)
            widths.append(32)
            aggs.append(_sc_aggregate(t2, src_r, dst_r))

        tr = params[f"transition{bi + 1}"]
        cout_real = tr["Wrel"].shape[0]
        if bi == 0:
            cout_pad = 160
            wa = _split_weight(_pad_mat(tr["Wrel"].T, sum(widths), cout_pad),
                               widths)
            wb = _split_weight(_pad_mat(tr["Wroot"].T, sum(widths), cout_pad),
                               widths)
            yt, st, sst = _conv_mm(
                pairs=list(zip(aggs, wa)),
                feats=list(zip(feats, wb)),
                directs=[],
                bias=_pad_cols(tr["brel"], cout_pad),
                cout=cout_pad,
            )
            chunks = _bn_relu(yt, st, sst,
                              _pad_cols(tr["gamma"], cout_pad),
                              _pad_cols(tr["beta"], cout_pad),
                              nchunks=cout_pad // _C)
            feats = list(chunks)
            widths = [32, 32, 32, 32, 16]
            aggs = [_sc_aggregate(ch, src_r, dst_r) for ch in chunks]
        else:
            cout_pad = 208
            wa = _split_weight(_pad_mat(tr["Wrel"].T, sum(widths), cout_pad),
                               widths)
            wb = _split_weight(_pad_mat(tr["Wroot"].T, sum(widths), cout_pad),
                               widths)
            yt, st, sst = _conv_mm(
                pairs=list(zip(aggs, wa)),
                feats=list(zip(feats, wb)),
                directs=[],
                bias=_pad_cols(tr["brel"], cout_pad),
                cout=cout_pad,
            )
            (hfin,) = _bn_relu(yt, st, sst,
                               _pad_cols(tr["gamma"], cout_pad),
                               _pad_cols(tr["beta"], cout_pad),
                               mask_col=cout_real)

    return _pool_cls(hfin, batch_r, params["cls_W"].T, params["cls_b"])
